# baseline (speedup 1.0000x reference)
import functools
import math

import jax
import jax.numpy as jnp
from jax import lax
from jax.experimental import pallas as pl
from jax.experimental.pallas import tpu as pltpu
from jax.experimental.pallas import tpu_sc as plsc

_TEMPERATURE = math.e
_P_BLK = 512


def _gating_kernel(k_active, x_ref, gw_ref, gb_ref, w_ref, m_ref):
    x = x_ref[...]
    gw = gw_ref[...]
    gb = gb_ref[...]
    e = gw.shape[1]
    scores = (jnp.dot(x, gw, preferred_element_type=jnp.float32) + gb) / _TEMPERATURE
    scores = scores - jnp.max(scores, axis=-1, keepdims=True)
    ex = jnp.exp(scores)
    probs = ex / jnp.sum(ex, axis=-1, keepdims=True)
    pi = probs[:, :, None]
    pj = probs[:, None, :]
    ii = lax.broadcasted_iota(jnp.int32, (1, e, e), 1)
    jj = lax.broadcasted_iota(jnp.int32, (1, e, e), 2)
    beats = jnp.logical_or(pj > pi, jnp.logical_and(pj == pi, jj < ii))
    rank = jnp.sum(beats.astype(jnp.int32), axis=2)
    mask = (rank < k_active).astype(jnp.float32)
    w = probs * mask
    w_ref[...] = w / (jnp.sum(w, axis=-1, keepdims=True) + 1e-8)
    m_ref[...] = mask


def _grouped_mm_kernel(be_ref, bv_ref, xs_ref, W1_ref, b1_ref, W2_ref,
                       b2_ref, out_ref):
    i = pl.program_id(0)

    @pl.when(bv_ref[i] == 1)
    def _compute():
        w1 = W1_ref[0]
        w2 = W2_ref[0]
        b1 = b1_ref[0]
        b2 = b2_ref[0]
        bt = xs_ref.shape[0]
        hb = bt // 2
        for half in range(2):
            sl = pl.ds(half * hb, hb)
            x_blk = xs_ref[sl, :]
            h = jnp.dot(x_blk, w1, preferred_element_type=jnp.float32)
            h = jnp.maximum(h + b1, 0.0)
            o = jnp.dot(h, w2, preferred_element_type=jnp.float32)
            out_ref[sl, :] = o + b2


_SC_NC = 2
_SC_NS = 16
_SC_L = 16


def _make_sc_gather(P, D, T):
    NC, NS = _SC_NC, _SC_NS
    NW = NC * NS
    per_w = P // NW
    CH = next(c for c in (64, 32, 16, 8) if per_w % c == 0)
    n_ch = per_w // CH
    mesh = plsc.VectorSubcoreMesh(core_axis_name="c", subcore_axis_name="s")

    @functools.partial(
        pl.kernel, mesh=mesh,
        out_type=jax.ShapeDtypeStruct((P, D), jnp.float32),
        scratch_types=[
            pltpu.VMEM((CH,), jnp.int32),
            pltpu.VMEM((CH, D), jnp.float32),
            pltpu.SemaphoreType.DMA,
        ],
    )
    def gather_k(x_hbm, idx_hbm, out_hbm, idx_v, rows_v, sem):
        wid = lax.axis_index("s") * NC + lax.axis_index("c")
        base = wid * per_w

        def body(ci, carry):
            off = base + ci * CH
            pltpu.sync_copy(idx_hbm.at[pl.ds(off, CH)], idx_v)
            pltpu.async_copy(x_hbm.at[idx_v], rows_v, sem).wait()
            pltpu.sync_copy(rows_v, out_hbm.at[pl.ds(off, CH)])
            return carry

        lax.fori_loop(0, n_ch, body, 0)

    return gather_k


def _make_sc_combine(P, O, T, k_active):
    NC, NS, L = _SC_NC, _SC_NS, _SC_L
    NW = NC * NS
    tok_per_w = T // NW
    CT = min(16, tok_per_w)
    n_ch = tok_per_w // CT
    NPAIR = CT * k_active
    n_lane_ch = O // L
    mesh = plsc.VectorSubcoreMesh(core_axis_name="c", subcore_axis_name="s")

    @functools.partial(
        pl.kernel, mesh=mesh,
        out_type=jax.ShapeDtypeStruct((T, O), jnp.float32),
        scratch_types=[
            pltpu.VMEM((NPAIR,), jnp.int32),
            pltpu.VMEM((NPAIR, O), jnp.float32),
            pltpu.VMEM((NPAIR, 128), jnp.float32),
            pltpu.VMEM((CT, O), jnp.float32),
            pltpu.SemaphoreType.DMA,
            pltpu.SemaphoreType.DMA,
        ],
    )
    def combine_k(outp_hbm, wexp_hbm, pos5_hbm, y_hbm, idx_v, rows_v, w_v,
                  y_v, sem1, sem2):
        wid = lax.axis_index("s") * NC + lax.axis_index("c")
        tok_base = wid * tok_per_w

        def chunk_body(ci, carry):
            tok0 = tok_base + ci * CT
            pltpu.sync_copy(pos5_hbm.at[pl.ds(tok0 * k_active, NPAIR)], idx_v)
            cp1 = pltpu.async_copy(outp_hbm.at[idx_v], rows_v, sem1)
            cp2 = pltpu.async_copy(wexp_hbm.at[idx_v], w_v, sem2)
            cp1.wait()
            cp2.wait()

            def tok_body(j, carry2):
                def lane_body(c, carry3):
                    sl = pl.ds(c * L, L)
                    wsl = pl.ds(0, L)
                    acc = w_v[j * k_active, wsl] * rows_v[j * k_active, sl]
                    for k in range(1, k_active):
                        acc = acc + w_v[j * k_active + k, wsl] * rows_v[j * k_active + k, sl]
                    y_v[j, sl] = acc
                    return carry3

                lax.fori_loop(0, n_lane_ch, lane_body, 0)
                return carry2

            lax.fori_loop(0, CT, tok_body, 0)
            pltpu.sync_copy(y_v, y_hbm.at[pl.ds(tok0, CT)])
            return carry

        lax.fori_loop(0, n_ch, chunk_body, 0)

    return combine_k


def kernel(x, gate_W, gate_b, W1, b1, W2, b2):
    T, D = x.shape
    E = gate_W.shape[1]
    H = W1.shape[2]
    O = W2.shape[2]
    k_active = max(1, int(E * 0.7))
    n_blocks = (T * k_active + E * (_P_BLK - 1)) // _P_BLK + 1
    P = n_blocks * _P_BLK

    bt_gate = min(T, 512)
    weights, maskf = pl.pallas_call(
        functools.partial(_gating_kernel, k_active),
        grid=(T // bt_gate,),
        in_specs=[
            pl.BlockSpec((bt_gate, D), lambda t: (t, 0)),
            pl.BlockSpec((D, E), lambda t: (0, 0)),
            pl.BlockSpec((1, E), lambda t: (0, 0)),
        ],
        out_specs=[
            pl.BlockSpec((bt_gate, E), lambda t: (t, 0)),
            pl.BlockSpec((bt_gate, E), lambda t: (t, 0)),
        ],
        out_shape=[
            jax.ShapeDtypeStruct((T, E), jnp.float32),
            jax.ShapeDtypeStruct((T, E), jnp.float32),
        ],
    )(x, gate_W, gate_b.reshape(1, E))

    m = maskf.astype(jnp.int32)
    incl = jnp.cumsum(m, axis=0)
    rank_e = incl - m
    counts = incl[-1]
    padded = ((counts + _P_BLK - 1) // _P_BLK) * _P_BLK
    cp = jnp.cumsum(padded)
    offs = cp - padded
    pos = offs[None, :] + rank_e
    pos_sc = jnp.where(m == 1, pos, P).reshape(-1)
    ids = lax.broadcasted_iota(jnp.int32, (T, E), 0).reshape(-1)
    tok_idx = jnp.zeros((P,), jnp.int32).at[pos_sc].set(ids, mode="drop")
    w_flat = jnp.zeros((P,), jnp.float32).at[pos_sc].set(
        weights.reshape(-1), mode="drop")
    w_exp = jnp.broadcast_to(w_flat[:, None], (P, 128)) + jnp.zeros(
        (P, 128), jnp.float32)
    bstart = jnp.arange(n_blocks, dtype=jnp.int32) * _P_BLK
    nfull = jnp.sum((bstart[:, None] >= cp[None, :]).astype(jnp.int32), axis=1)
    block_valid = (bstart < cp[E - 1]).astype(jnp.int32)
    block_expert = jnp.minimum(nfull, E - 1)
    ra = jnp.cumsum(m, axis=1) - m
    pos5 = jnp.stack(
        [jnp.sum(jnp.where((ra == k) & (m == 1), pos, 0), axis=1)
         for k in range(k_active)], axis=1).astype(jnp.int32)
    pos5_flat = pos5.reshape(-1)

    x_sorted = _make_sc_gather(P, D, T)(x, tok_idx)

    out_pairs = pl.pallas_call(
        _grouped_mm_kernel,
        grid_spec=pltpu.PrefetchScalarGridSpec(
            num_scalar_prefetch=2,
            grid=(n_blocks,),
            in_specs=[
                pl.BlockSpec((_P_BLK, D), lambda i, be, bv: (i, 0)),
                pl.BlockSpec((1, D, H), lambda i, be, bv: (be[i], 0, 0)),
                pl.BlockSpec((1, 1, H), lambda i, be, bv: (be[i], 0, 0)),
                pl.BlockSpec((1, H, O), lambda i, be, bv: (be[i], 0, 0)),
                pl.BlockSpec((1, 1, O), lambda i, be, bv: (be[i], 0, 0)),
            ],
            out_specs=pl.BlockSpec((_P_BLK, O), lambda i, be, bv: (i, 0)),
        ),
        out_shape=jax.ShapeDtypeStruct((P, O), jnp.float32),
    )(block_expert, block_valid, x_sorted, W1, b1.reshape(E, 1, H), W2,
      b2.reshape(E, 1, O))

    y = _make_sc_combine(P, O, T, k_active)(out_pairs, w_exp, pos5_flat)
    return y


# speedup vs baseline: 1.0326x; 1.0326x over previous
import functools
import math

import jax
import jax.numpy as jnp
from jax import lax
from jax.experimental import pallas as pl
from jax.experimental.pallas import tpu as pltpu
from jax.experimental.pallas import tpu_sc as plsc

_TEMPERATURE = math.e
_P_BLK = 512


def _gating_kernel(k_active, x_ref, gw_ref, gb_ref, w_ref, m_ref):
    x = x_ref[...]
    gw = gw_ref[...]
    gb = gb_ref[...]
    e = gw.shape[1]
    scores = (jnp.dot(x, gw, preferred_element_type=jnp.float32) + gb) / _TEMPERATURE
    scores = scores - jnp.max(scores, axis=-1, keepdims=True)
    ex = jnp.exp(scores)
    probs = ex / jnp.sum(ex, axis=-1, keepdims=True)
    pi = probs[:, :, None]
    pj = probs[:, None, :]
    ii = lax.broadcasted_iota(jnp.int32, (1, e, e), 1)
    jj = lax.broadcasted_iota(jnp.int32, (1, e, e), 2)
    beats = jnp.logical_or(pj > pi, jnp.logical_and(pj == pi, jj < ii))
    rank = jnp.sum(beats.astype(jnp.int32), axis=2)
    mask = (rank < k_active).astype(jnp.float32)
    w = probs * mask
    w_ref[...] = w / (jnp.sum(w, axis=-1, keepdims=True) + 1e-8)
    m_ref[...] = mask


def _grouped_mm_kernel(be_ref, bv_ref, xs_ref, W1_ref, b1_ref, W2_ref,
                       b2_ref, out_ref):
    i = pl.program_id(0)

    @pl.when(bv_ref[i] == 1)
    def _compute():
        w1 = W1_ref[0]
        w2 = W2_ref[0]
        b1 = b1_ref[0]
        b2 = b2_ref[0]
        bt = xs_ref.shape[0]
        hb = bt // 2
        for half in range(2):
            sl = pl.ds(half * hb, hb)
            x_blk = xs_ref[sl, :]
            h = jnp.dot(x_blk, w1, preferred_element_type=jnp.float32)
            h = jnp.maximum(h + b1, 0.0)
            o = jnp.dot(h, w2, preferred_element_type=jnp.float32)
            out_ref[sl, :] = o + b2


_SC_NC = 2
_SC_NS = 16
_SC_L = 16


def _make_sc_gather(P, D, T):
    NC, NS = _SC_NC, _SC_NS
    NW = NC * NS
    per_w = P // NW
    CH = next(c for c in (48, 32, 16, 8) if per_w % c == 0)
    n_ch = per_w // CH
    mesh = plsc.VectorSubcoreMesh(core_axis_name="c", subcore_axis_name="s")

    @functools.partial(
        pl.kernel, mesh=mesh,
        out_type=jax.ShapeDtypeStruct((P, D), jnp.float32),
        scratch_types=[
            pltpu.VMEM((per_w,), jnp.int32),
            pltpu.VMEM((CH, D), jnp.float32),
            pltpu.VMEM((CH, D), jnp.float32),
            pltpu.SemaphoreType.DMA,
            pltpu.SemaphoreType.DMA,
            pltpu.SemaphoreType.DMA,
            pltpu.SemaphoreType.DMA,
        ],
    )
    def gather_k(x_hbm, idx_hbm, out_hbm, idx_v, rows0, rows1, sg0, sg1,
                 ss0, ss1):
        wid = lax.axis_index("s") * NC + lax.axis_index("c")
        base = wid * per_w
        rows = (rows0, rows1)
        sg = (sg0, sg1)
        ss = (ss0, ss1)
        pltpu.sync_copy(idx_hbm.at[pl.ds(base, per_w)], idx_v)
        g_cp = [None, None]
        s_cp = [None, None]
        g_cp[0] = pltpu.async_copy(
            x_hbm.at[idx_v.at[pl.ds(0, CH)]], rows0, sg0)
        for ci in range(n_ch):
            b = ci % 2
            if ci + 1 < n_ch:
                nb = (ci + 1) % 2
                if s_cp[nb] is not None:
                    s_cp[nb].wait()
                g_cp[nb] = pltpu.async_copy(
                    x_hbm.at[idx_v.at[pl.ds((ci + 1) * CH, CH)]], rows[nb],
                    sg[nb])
            g_cp[b].wait()
            s_cp[b] = pltpu.async_copy(
                rows[b], out_hbm.at[pl.ds(base + ci * CH, CH)], ss[b])
        for b in range(2):
            if s_cp[b] is not None:
                s_cp[b].wait()

    return gather_k


def _make_sc_combine(P, O, T, k_active):
    NC, NS, L = _SC_NC, _SC_NS, _SC_L
    NW = NC * NS
    tok_per_w = T // NW
    CT = min(8, tok_per_w)
    n_ch = tok_per_w // CT
    NPAIR = CT * k_active
    n_lane_ch = O // L
    mesh = plsc.VectorSubcoreMesh(core_axis_name="c", subcore_axis_name="s")

    @functools.partial(
        pl.kernel, mesh=mesh,
        out_type=jax.ShapeDtypeStruct((T, O), jnp.float32),
        scratch_types=[
            pltpu.VMEM((tok_per_w * k_active,), jnp.int32),
            pltpu.VMEM((NPAIR, O), jnp.float32),
            pltpu.VMEM((NPAIR, O), jnp.float32),
            pltpu.VMEM((NPAIR, 128), jnp.float32),
            pltpu.VMEM((NPAIR, 128), jnp.float32),
            pltpu.VMEM((CT, O), jnp.float32),
            pltpu.VMEM((CT, O), jnp.float32),
            pltpu.SemaphoreType.DMA,
            pltpu.SemaphoreType.DMA,
            pltpu.SemaphoreType.DMA,
            pltpu.SemaphoreType.DMA,
            pltpu.SemaphoreType.DMA,
            pltpu.SemaphoreType.DMA,
        ],
    )
    def combine_k(outp_hbm, wexp_hbm, pos5_hbm, y_hbm, idx_v, rows0, rows1,
                  w0, w1, y0, y1, sr0, sr1, sw0, sw1, sy0, sy1):
        wid = lax.axis_index("s") * NC + lax.axis_index("c")
        tok_base = wid * tok_per_w
        rows = (rows0, rows1)
        wv = (w0, w1)
        yv = (y0, y1)
        sr = (sr0, sr1)
        sw = (sw0, sw1)
        sy = (sy0, sy1)
        pltpu.sync_copy(
            pos5_hbm.at[pl.ds(tok_base * k_active, tok_per_w * k_active)],
            idx_v)
        r_cp = [None, None]
        w_cp = [None, None]
        y_cp = [None, None]

        def start(ci, b):
            isl = idx_v.at[pl.ds(ci * NPAIR, NPAIR)]
            r_cp[b] = pltpu.async_copy(outp_hbm.at[isl], rows[b], sr[b])
            w_cp[b] = pltpu.async_copy(wexp_hbm.at[isl], wv[b], sw[b])

        start(0, 0)
        for ci in range(n_ch):
            b = ci % 2
            if ci + 1 < n_ch:
                nb = (ci + 1) % 2
                if y_cp[nb] is not None:
                    y_cp[nb].wait()
                start(ci + 1, nb)
            r_cp[b].wait()
            w_cp[b].wait()
            rows_v = rows[b]
            w_v = wv[b]
            y_v = yv[b]

            def tok_body(j, carry2):
                def lane_body(c, carry3):
                    sl = pl.ds(c * L, L)
                    wsl = pl.ds(0, L)
                    acc = w_v[j * k_active, wsl] * rows_v[j * k_active, sl]
                    for k in range(1, k_active):
                        acc = acc + w_v[j * k_active + k, wsl] * rows_v[j * k_active + k, sl]
                    y_v[j, sl] = acc
                    return carry3

                lax.fori_loop(0, n_lane_ch, lane_body, 0)
                return carry2

            lax.fori_loop(0, CT, tok_body, 0)
            y_cp[b] = pltpu.async_copy(
                y_v, y_hbm.at[pl.ds(tok_base + ci * CT, CT)], sy[b])
        for b in range(2):
            if y_cp[b] is not None:
                y_cp[b].wait()

    return combine_k


def kernel(x, gate_W, gate_b, W1, b1, W2, b2):
    T, D = x.shape
    E = gate_W.shape[1]
    H = W1.shape[2]
    O = W2.shape[2]
    k_active = max(1, int(E * 0.7))
    n_blocks = (T * k_active + E * (_P_BLK - 1)) // _P_BLK + 1
    P = n_blocks * _P_BLK

    bt_gate = min(T, 512)
    weights, maskf = pl.pallas_call(
        functools.partial(_gating_kernel, k_active),
        grid=(T // bt_gate,),
        in_specs=[
            pl.BlockSpec((bt_gate, D), lambda t: (t, 0)),
            pl.BlockSpec((D, E), lambda t: (0, 0)),
            pl.BlockSpec((1, E), lambda t: (0, 0)),
        ],
        out_specs=[
            pl.BlockSpec((bt_gate, E), lambda t: (t, 0)),
            pl.BlockSpec((bt_gate, E), lambda t: (t, 0)),
        ],
        out_shape=[
            jax.ShapeDtypeStruct((T, E), jnp.float32),
            jax.ShapeDtypeStruct((T, E), jnp.float32),
        ],
    )(x, gate_W, gate_b.reshape(1, E))

    m = maskf.astype(jnp.int32)
    incl = jnp.cumsum(m, axis=0)
    rank_e = incl - m
    counts = incl[-1]
    padded = ((counts + _P_BLK - 1) // _P_BLK) * _P_BLK
    cp = jnp.cumsum(padded)
    offs = cp - padded
    pos = offs[None, :] + rank_e
    pos_sc = jnp.where(m == 1, pos, P).reshape(-1)
    ids = lax.broadcasted_iota(jnp.int32, (T, E), 0).reshape(-1)
    tok_idx = jnp.zeros((P,), jnp.int32).at[pos_sc].set(ids, mode="drop")
    w_flat = jnp.zeros((P,), jnp.float32).at[pos_sc].set(
        weights.reshape(-1), mode="drop")
    w_exp = jnp.broadcast_to(w_flat[:, None], (P, 128)) + jnp.zeros(
        (P, 128), jnp.float32)
    bstart = jnp.arange(n_blocks, dtype=jnp.int32) * _P_BLK
    nfull = jnp.sum((bstart[:, None] >= cp[None, :]).astype(jnp.int32), axis=1)
    block_valid = (bstart < cp[E - 1]).astype(jnp.int32)
    block_expert = jnp.minimum(nfull, E - 1)
    ra = jnp.cumsum(m, axis=1) - m
    pos5 = jnp.stack(
        [jnp.sum(jnp.where((ra == k) & (m == 1), pos, 0), axis=1)
         for k in range(k_active)], axis=1).astype(jnp.int32)
    pos5_flat = pos5.reshape(-1)

    x_sorted = _make_sc_gather(P, D, T)(x, tok_idx)

    out_pairs = pl.pallas_call(
        _grouped_mm_kernel,
        grid_spec=pltpu.PrefetchScalarGridSpec(
            num_scalar_prefetch=2,
            grid=(n_blocks,),
            in_specs=[
                pl.BlockSpec((_P_BLK, D), lambda i, be, bv: (i, 0)),
                pl.BlockSpec((1, D, H), lambda i, be, bv: (be[i], 0, 0)),
                pl.BlockSpec((1, 1, H), lambda i, be, bv: (be[i], 0, 0)),
                pl.BlockSpec((1, H, O), lambda i, be, bv: (be[i], 0, 0)),
                pl.BlockSpec((1, 1, O), lambda i, be, bv: (be[i], 0, 0)),
            ],
            out_specs=pl.BlockSpec((_P_BLK, O), lambda i, be, bv: (i, 0)),
        ),
        out_shape=jax.ShapeDtypeStruct((P, O), jnp.float32),
    )(block_expert, block_valid, x_sorted, W1, b1.reshape(E, 1, H), W2,
      b2.reshape(E, 1, O))

    y = _make_sc_combine(P, O, T, k_active)(out_pairs, w_exp, pos5_flat)
    return y


# speedup vs baseline: 2.0853x; 2.0194x over previous
import functools
import math

import jax
import jax.numpy as jnp
from jax import lax
from jax.experimental import pallas as pl
from jax.experimental.pallas import tpu as pltpu

_TEMPERATURE = math.e


def _gating_kernel(k_active, x_ref, gw_ref, gb_ref, w_ref):
    x = x_ref[...]
    gw = gw_ref[...]
    gb = gb_ref[...]
    e = gw.shape[1]
    scores = (jnp.dot(x, gw, preferred_element_type=jnp.float32) + gb) / _TEMPERATURE
    scores = scores - jnp.max(scores, axis=-1, keepdims=True)
    ex = jnp.exp(scores)
    probs = ex / jnp.sum(ex, axis=-1, keepdims=True)
    pi = probs[:, :, None]
    pj = probs[:, None, :]
    ii = lax.broadcasted_iota(jnp.int32, (1, e, e), 1)
    jj = lax.broadcasted_iota(jnp.int32, (1, e, e), 2)
    beats = jnp.logical_or(pj > pi, jnp.logical_and(pj == pi, jj < ii))
    rank = jnp.sum(beats.astype(jnp.int32), axis=2)
    mask = (rank < k_active).astype(jnp.float32)
    w = probs * mask
    w_ref[...] = w / (jnp.sum(w, axis=-1, keepdims=True) + 1e-8)


def _moe_kernel(bt, x_ref, w_ref, W1_ref, b1_ref, W2_ref, b2_ref, out_ref):
    e = pl.program_id(0)
    t = pl.program_id(1)
    w1 = W1_ref[0]
    w2 = W2_ref[0]
    b1 = b1_ref[0]
    n_e = w_ref.shape[1]
    onehot = (lax.broadcasted_iota(jnp.int32, (1, n_e), 1) == e).astype(jnp.float32)

    hb = bt // 2
    parts = []
    for i in range(2):
        sl = pl.ds(t * bt + i * hb, hb)
        x_blk = x_ref[pl.ds(i * hb, hb), :]
        h = jnp.dot(x_blk, w1, preferred_element_type=jnp.float32)
        h = jnp.maximum(h + b1, 0.0)
        o = jnp.dot(h.astype(jnp.bfloat16), w2,
                    preferred_element_type=jnp.float32)
        w_blk = w_ref[sl, :]
        w_col = jnp.sum(w_blk * onehot, axis=1, keepdims=True)
        parts.append((sl, w_blk, w_col * o))

    @pl.when(e == 0)
    def _init():
        for sl, w_blk, contrib in parts:
            out_ref[sl, :] = contrib + jnp.dot(
                w_blk, b2_ref[...], preferred_element_type=jnp.float32)

    @pl.when(e > 0)
    def _acc():
        for sl, _, contrib in parts:
            out_ref[sl, :] = out_ref[sl, :] + contrib


def kernel(x, gate_W, gate_b, W1, b1, W2, b2):
    T, D = x.shape
    E = gate_W.shape[1]
    H = W1.shape[2]
    O = W2.shape[2]
    k_active = max(1, int(E * 0.7))

    bt_gate = min(T, 512)
    weights = pl.pallas_call(
        functools.partial(_gating_kernel, k_active),
        grid=(T // bt_gate,),
        in_specs=[
            pl.BlockSpec((bt_gate, D), lambda t: (t, 0)),
            pl.BlockSpec((D, E), lambda t: (0, 0)),
            pl.BlockSpec((1, E), lambda t: (0, 0)),
        ],
        out_specs=pl.BlockSpec((bt_gate, E), lambda t: (t, 0)),
        out_shape=jax.ShapeDtypeStruct((T, E), jnp.float32),
    )(x, gate_W, gate_b.reshape(1, E))

    xb = x.astype(jnp.bfloat16)
    W1b = W1.astype(jnp.bfloat16)
    W2b = W2.astype(jnp.bfloat16)

    bt = min(T, 512)
    n_bt = T // bt
    y = pl.pallas_call(
        functools.partial(_moe_kernel, bt),
        grid=(E, n_bt),
        in_specs=[
            pl.BlockSpec((bt, D), lambda e, t: (t, 0)),
            pl.BlockSpec((T, E), lambda e, t: (0, 0)),
            pl.BlockSpec((1, D, H), lambda e, t: (e, 0, 0)),
            pl.BlockSpec((1, 1, H), lambda e, t: (e, 0, 0)),
            pl.BlockSpec((1, H, O), lambda e, t: (e, 0, 0)),
            pl.BlockSpec((E, O), lambda e, t: (0, 0)),
        ],
        out_specs=pl.BlockSpec((T, O), lambda e, t: (0, 0)),
        out_shape=jax.ShapeDtypeStruct((T, O), jnp.float32),
        compiler_params=pltpu.CompilerParams(vmem_limit_bytes=112 * 1024 * 1024),
    )(xb, weights, W1b, b1.reshape(E, 1, H), W2b, b2)
    return y


# speedup vs baseline: 2.3709x; 1.1370x over previous
import functools
import math

import jax
import jax.numpy as jnp
from jax import lax
from jax.experimental import pallas as pl
from jax.experimental.pallas import tpu as pltpu

_TEMPERATURE = math.e


def _gating_kernel(k_active, x_ref, gw_ref, gb_ref, w_ref):
    x = x_ref[...]
    gw = gw_ref[...]
    gb = gb_ref[...]
    e = gw.shape[1]
    scores = (jnp.dot(x, gw, preferred_element_type=jnp.float32) + gb) / _TEMPERATURE
    scores = scores - jnp.max(scores, axis=-1, keepdims=True)
    ex = jnp.exp(scores)
    probs = ex / jnp.sum(ex, axis=-1, keepdims=True)
    pi = probs[:, :, None]
    pj = probs[:, None, :]
    ii = lax.broadcasted_iota(jnp.int32, (1, e, e), 1)
    jj = lax.broadcasted_iota(jnp.int32, (1, e, e), 2)
    beats = jnp.logical_or(pj > pi, jnp.logical_and(pj == pi, jj < ii))
    rank = jnp.sum(beats.astype(jnp.int32), axis=2)
    mask = (rank < k_active).astype(jnp.float32)
    w = probs * mask
    w_ref[...] = w / (jnp.sum(w, axis=-1, keepdims=True) + 1e-8)


def _moe_kernel(bt, x_ref, w_ref, W1_ref, b1_ref, W2_ref, b2_ref, out_ref):
    e = pl.program_id(0)
    t = pl.program_id(1)
    hc = pl.program_id(2)
    w1 = W1_ref[0]
    w2 = W2_ref[0]
    b1 = b1_ref[0]
    n_e = w_ref.shape[1]
    onehot = (lax.broadcasted_iota(jnp.int32, (1, n_e), 1) == e).astype(jnp.float32)

    hb = bt // 2
    parts = []
    for i in range(2):
        sl = pl.ds(t * bt + i * hb, hb)
        x_blk = x_ref[pl.ds(i * hb, hb), :]
        h = jnp.dot(x_blk, w1, preferred_element_type=jnp.float32)
        h = jnp.maximum(h + b1, 0.0)
        o = jnp.dot(h, w2, preferred_element_type=jnp.float32)
        w_blk = w_ref[sl, :]
        w_col = jnp.sum(w_blk * onehot, axis=1, keepdims=True)
        parts.append((sl, w_blk, w_col * o))

    first = jnp.logical_and(e == 0, hc == 0)

    @pl.when(first)
    def _init():
        for sl, w_blk, contrib in parts:
            out_ref[sl, :] = contrib + jnp.dot(
                w_blk, b2_ref[...], preferred_element_type=jnp.float32)

    @pl.when(jnp.logical_not(first))
    def _acc():
        for sl, _, contrib in parts:
            out_ref[sl, :] = out_ref[sl, :] + contrib


def kernel(x, gate_W, gate_b, W1, b1, W2, b2):
    T, D = x.shape
    E = gate_W.shape[1]
    H = W1.shape[2]
    O = W2.shape[2]
    k_active = max(1, int(E * 0.7))

    bt_gate = min(T, 512)
    weights = pl.pallas_call(
        functools.partial(_gating_kernel, k_active),
        grid=(T // bt_gate,),
        in_specs=[
            pl.BlockSpec((bt_gate, D), lambda t: (t, 0)),
            pl.BlockSpec((D, E), lambda t: (0, 0)),
            pl.BlockSpec((1, E), lambda t: (0, 0)),
        ],
        out_specs=pl.BlockSpec((bt_gate, E), lambda t: (t, 0)),
        out_shape=jax.ShapeDtypeStruct((T, E), jnp.float32),
    )(x, gate_W, gate_b.reshape(1, E))

    bt = min(T, 1024)
    n_bt = T // bt
    hcs = min(H, 1024)
    n_hc = H // hcs
    y = pl.pallas_call(
        functools.partial(_moe_kernel, bt),
        grid=(E, n_bt, n_hc),
        in_specs=[
            pl.BlockSpec((bt, D), lambda e, t, h: (t, 0)),
            pl.BlockSpec((T, E), lambda e, t, h: (0, 0)),
            pl.BlockSpec((1, D, hcs), lambda e, t, h: (e, 0, h)),
            pl.BlockSpec((1, 1, hcs), lambda e, t, h: (e, 0, h)),
            pl.BlockSpec((1, hcs, O), lambda e, t, h: (e, h, 0)),
            pl.BlockSpec((E, O), lambda e, t, h: (0, 0)),
        ],
        out_specs=pl.BlockSpec((T, O), lambda e, t, h: (0, 0)),
        out_shape=jax.ShapeDtypeStruct((T, O), jnp.float32),
        compiler_params=pltpu.CompilerParams(vmem_limit_bytes=60 * 1024 * 1024),
    )(x, weights, W1, b1.reshape(E, 1, H), W2, b2)
    return y


# speedup vs baseline: 2.4307x; 1.0252x over previous
import functools
import math

import jax
import jax.numpy as jnp
from jax import lax
from jax.experimental import pallas as pl
from jax.experimental.pallas import tpu as pltpu

_TEMPERATURE = math.e


def _gating_kernel(k_active, x_ref, gw_ref, gb_ref, w_ref):
    x = x_ref[...]
    gw = gw_ref[...]
    gb = gb_ref[...]
    e = gw.shape[1]
    scores = (jnp.dot(x, gw, preferred_element_type=jnp.float32) + gb) / _TEMPERATURE
    scores = scores - jnp.max(scores, axis=-1, keepdims=True)
    ex = jnp.exp(scores)
    probs = ex / jnp.sum(ex, axis=-1, keepdims=True)
    ivec = lax.broadcasted_iota(jnp.int32, (1, e), 1)
    rank = jnp.zeros(probs.shape, jnp.int32)
    for j in range(e):
        pj = probs[:, j:j + 1]
        beats = jnp.logical_or(pj > probs,
                               jnp.logical_and(pj == probs, j < ivec))
        rank = rank + beats.astype(jnp.int32)
    mask = (rank < k_active).astype(jnp.float32)
    w = probs * mask
    w_ref[...] = w / (jnp.sum(w, axis=-1, keepdims=True) + 1e-8)


def _moe_kernel(bt, x_ref, w_ref, W1_ref, b1_ref, W2_ref, b2_ref, out_ref):
    e = pl.program_id(0)
    t = pl.program_id(1)
    w1 = W1_ref[0]
    w2 = W2_ref[0]
    b1 = b1_ref[0]
    n_e = w_ref.shape[1]
    onehot = (lax.broadcasted_iota(jnp.int32, (1, n_e), 1) == e).astype(jnp.float32)

    hb = bt // 2
    parts = []
    for i in range(2):
        sl = pl.ds(t * bt + i * hb, hb)
        x_blk = x_ref[pl.ds(i * hb, hb), :]
        h = jnp.dot(x_blk, w1, preferred_element_type=jnp.float32)
        h = jnp.maximum(h + b1, 0.0)
        o = jnp.dot(h, w2, preferred_element_type=jnp.float32)
        w_blk = w_ref[sl, :]
        w_col = jnp.sum(w_blk * onehot, axis=1, keepdims=True)
        parts.append((sl, w_blk, w_col * o))

    @pl.when(e == 0)
    def _init():
        for sl, w_blk, contrib in parts:
            out_ref[sl, :] = contrib + jnp.dot(
                w_blk, b2_ref[...], preferred_element_type=jnp.float32)

    @pl.when(e > 0)
    def _acc():
        for sl, _, contrib in parts:
            out_ref[sl, :] = out_ref[sl, :] + contrib


def kernel(x, gate_W, gate_b, W1, b1, W2, b2):
    T, D = x.shape
    E = gate_W.shape[1]
    H = W1.shape[2]
    O = W2.shape[2]
    k_active = max(1, int(E * 0.7))

    bt_gate = min(T, 512)
    weights = pl.pallas_call(
        functools.partial(_gating_kernel, k_active),
        grid=(T // bt_gate,),
        in_specs=[
            pl.BlockSpec((bt_gate, D), lambda t: (t, 0)),
            pl.BlockSpec((D, E), lambda t: (0, 0)),
            pl.BlockSpec((1, E), lambda t: (0, 0)),
        ],
        out_specs=pl.BlockSpec((bt_gate, E), lambda t: (t, 0)),
        out_shape=jax.ShapeDtypeStruct((T, E), jnp.float32),
    )(x, gate_W, gate_b.reshape(1, E))

    bt = min(T, 512)
    n_bt = T // bt
    y = pl.pallas_call(
        functools.partial(_moe_kernel, bt),
        grid=(E, n_bt),
        in_specs=[
            pl.BlockSpec((bt, D), lambda e, t: (t, 0)),
            pl.BlockSpec((T, E), lambda e, t: (0, 0)),
            pl.BlockSpec((1, D, H), lambda e, t: (e, 0, 0)),
            pl.BlockSpec((1, 1, H), lambda e, t: (e, 0, 0)),
            pl.BlockSpec((1, H, O), lambda e, t: (e, 0, 0)),
            pl.BlockSpec((E, O), lambda e, t: (0, 0)),
        ],
        out_specs=pl.BlockSpec((T, O), lambda e, t: (0, 0)),
        out_shape=jax.ShapeDtypeStruct((T, O), jnp.float32),
        compiler_params=pltpu.CompilerParams(vmem_limit_bytes=112 * 1024 * 1024),
    )(x, weights, W1, b1.reshape(E, 1, H), W2, b2)
    return y
